# baseline (speedup 1.0000x reference)
import functools

import jax
import jax.numpy as jnp
import numpy as np
from jax import lax
from jax.experimental import pallas as pl
from jax.experimental.pallas import tpu as pltpu
from jax.experimental.pallas import tpu_sc as plsc

N, E, F_IN, H, C, CH = 10000, 320000, 128, 16, 10, 3
NC, NS = 2, 16
NW = NC * NS
NPAD = 10240
ROWS_PER_SUB = NPAD // NS
F = 16
EPW = E // NW
K = 125
STEPS = EPW // K
NBUF = 8
STD = float(np.sqrt(2.0 / (N + 1)))

_mesh = plsc.VectorSubcoreMesh(core_axis_name="c", subcore_axis_name="s")


@functools.partial(
    pl.kernel,
    out_type=jax.ShapeDtypeStruct((NC, NPAD), jnp.float32),
    mesh=_mesh,
    scratch_types=[
        pltpu.VMEM((STEPS, K), jnp.int32),
        pltpu.VMEM((128,), jnp.float32),
        pltpu.VMEM((ROWS_PER_SUB,), jnp.float32),
        pltpu.VMEM_SHARED((NPAD,), jnp.float32),
        pltpu.SemaphoreType.DMA,
    ],
    compiler_params=pltpu.CompilerParams(use_tc_tiling_on_sc=False),
)
def _deg_kernel(ei_hbm, out_hbm, didx, ones_v, zv, acc_s, sem):
    c = lax.axis_index("c")
    s = lax.axis_index("s")
    wid = s * NC + c

    ld = pltpu.async_copy(ei_hbm.at[1, wid], didx, sem)

    def fill_ones(i, carry):
        ones_v[pl.ds(i * 16, 16)] = jnp.ones((16,), jnp.float32)
        return carry

    lax.fori_loop(0, 128 // 16, fill_ones, 0)

    def fill_zero(i, carry):
        zv[pl.ds(i * 16, 16)] = jnp.zeros((16,), jnp.float32)
        return carry

    lax.fori_loop(0, ROWS_PER_SUB // 16, fill_zero, 0)
    pltpu.sync_copy(zv, acc_s.at[pl.ds(s * ROWS_PER_SUB, ROWS_PER_SUB)])
    ld.wait()
    plsc.subcore_barrier()

    def step(i, carry):
        pltpu.sync_copy(ones_v.at[pl.ds(0, K)], acc_s.at[didx.at[i]],
                        add=True)
        return carry

    lax.fori_loop(0, STEPS, step, 0)
    plsc.subcore_barrier()
    pltpu.sync_copy(acc_s.at[pl.ds(s * ROWS_PER_SUB, ROWS_PER_SUB)],
                    out_hbm.at[c, pl.ds(s * ROWS_PER_SUB, ROWS_PER_SUB)])


@functools.partial(
    pl.kernel,
    out_type=jax.ShapeDtypeStruct((NC, NPAD, F), jnp.float32),
    mesh=_mesh,
    scratch_types=[
        pltpu.VMEM((2, STEPS, K), jnp.int32),
        [pltpu.VMEM((K, F), jnp.float32) for _ in range(NBUF)],
        pltpu.VMEM((ROWS_PER_SUB, F), jnp.float32),
        pltpu.VMEM_SHARED((NPAD, F), jnp.float32),
        pltpu.VMEM_SHARED((NPAD, F), jnp.float32),
        [pltpu.SemaphoreType.DMA for _ in range(NBUF)],
        pltpu.SemaphoreType.DMA,
        pltpu.SemaphoreType.DMA,
    ],
    compiler_params=pltpu.CompilerParams(use_tc_tiling_on_sc=False),
)
def _segsum_kernel(g_hbm, ei_hbm, out_hbm,
                   eidx, rows, zv, acc_s, g_s, sems, isem, gsem):
    c = lax.axis_index("c")
    s = lax.axis_index("s")
    wid = s * NC + c

    ls = pltpu.async_copy(ei_hbm.at[0, wid], eidx.at[0], isem)
    row0 = s * ROWS_PER_SUB
    lg = pltpu.async_copy(g_hbm.at[pl.ds(row0, ROWS_PER_SUB)],
                          g_s.at[pl.ds(row0, ROWS_PER_SUB)], gsem)

    def fill_zero(i, carry):
        zv[i, :] = jnp.zeros((16,), jnp.float32)
        return carry

    lax.fori_loop(0, ROWS_PER_SUB, fill_zero, 0)
    pltpu.sync_copy(zv, acc_s.at[pl.ds(s * ROWS_PER_SUB, ROWS_PER_SUB)])
    ls.wait()
    pltpu.sync_copy(ei_hbm.at[1, wid], eidx.at[1])
    lg.wait()
    plsc.subcore_barrier()

    def batch(j, carry):
        descs = []
        for b in range(NBUF):
            descs.append(pltpu.async_copy(
                g_s.at[eidx.at[0, j * NBUF + b]], rows[b], sems[b]))
        for b in range(NBUF):
            descs[b].wait()
            pltpu.sync_copy(rows[b], acc_s.at[eidx.at[1, j * NBUF + b]],
                            add=True)
        return carry

    lax.fori_loop(0, STEPS // NBUF, batch, 0)
    plsc.subcore_barrier()
    pltpu.sync_copy(acc_s.at[pl.ds(s * ROWS_PER_SUB, ROWS_PER_SUB)],
                    out_hbm.at[c, pl.ds(s * ROWS_PER_SUB, ROWS_PER_SUB)])


def _tc_pre_body(x_ref, w1_ref, degp_ref, g_ref, dinv_ref):
    deg = degp_ref[0, :N, :] + degp_ref[1, :N, :] + 1.0
    dinv = lax.rsqrt(deg)
    dinv_ref[...] = dinv
    g_ref[:N, :] = jnp.dot(x_ref[...], w1_ref[...],
                           preferred_element_type=jnp.float32) * dinv
    g_ref[N:, :] = jnp.zeros((NPAD - N, F), jnp.float32)


def _tc_mid_body(accp_ref, g_ref, dinv_ref, b1_ref, w2p_ref, p_ref):
    acc = accp_ref[0, :N, :] + accp_ref[1, :N, :]
    dinv = dinv_ref[...]
    h = jnp.maximum(dinv * (acc + g_ref[:N, :]) + b1_ref[...], 0.0)
    p_ref[:N, :] = jnp.dot(h, w2p_ref[...],
                           preferred_element_type=jnp.float32) * dinv
    p_ref[N:, :] = jnp.zeros((NPAD - N, F), jnp.float32)


def _tc_head_body(accp_ref, p_ref, dinv_ref, b2_ref, cht_ref, c1w_ref,
                  c1b_ref, c2w_ref, c2b_ref, mlpw_ref, mlpb_ref, gam_ref,
                  bet_ref, out_ref):
    acc = accp_ref[0, :N, :] + accp_ref[1, :N, :]
    dinv = dinv_ref[...]
    z0 = (dinv * (acc + p_ref[:N, :]))[:, :C] + b2_ref[...]
    m = jnp.max(z0, axis=1, keepdims=True)
    lse = m + jnp.log(jnp.sum(jnp.exp(z0 - m), axis=1, keepdims=True))
    h0 = z0 - lse
    w0 = jnp.exp(cht_ref[:, 0:1] * STD)
    w1 = jnp.exp(cht_ref[:, 1:2] * STD)
    w2 = jnp.exp(cht_ref[:, 2:3] * STD)
    o0 = w0 * h0
    o1 = w1 * (jnp.dot(o0, c1w_ref[...],
                       preferred_element_type=jnp.float32) + c1b_ref[...])
    o01 = jnp.concatenate([o0, o1], axis=1)
    o2 = w2 * (jnp.dot(o01, c2w_ref[...],
                       preferred_element_type=jnp.float32) + c2b_ref[...])
    zc = jnp.concatenate([o01, o2], axis=1)
    z = jnp.dot(zc, mlpw_ref[...],
                preferred_element_type=jnp.float32) + mlpb_ref[...]
    mu = jnp.mean(z, axis=0, keepdims=True)
    var = jnp.mean((z - mu) ** 2, axis=0, keepdims=True)
    zn = (z - mu) * lax.rsqrt(var + 1e-5) * gam_ref[...] + bet_ref[...]
    m2 = jnp.max(zn, axis=1, keepdims=True)
    lse2 = m2 + jnp.log(jnp.sum(jnp.exp(zn - m2), axis=1, keepdims=True))
    out_ref[...] = zn - lse2


_tc_pre = pl.pallas_call(
    _tc_pre_body,
    out_shape=[jax.ShapeDtypeStruct((NPAD, H), jnp.float32),
               jax.ShapeDtypeStruct((N, 1), jnp.float32)],
)

_tc_mid = pl.pallas_call(
    _tc_mid_body,
    out_shape=jax.ShapeDtypeStruct((NPAD, F), jnp.float32),
)

_tc_head = pl.pallas_call(
    _tc_head_body,
    out_shape=jax.ShapeDtypeStruct((N, C), jnp.float32),
)


def kernel(x0, x1, x2, edge_index0, edge_index1, edge_index2, conv1_w,
           conv1_b, conv2_w, conv2_b, child1_w, child1_b, child2_w,
           child2_b, mlp_w, mlp_b, bn_gamma, bn_beta, ch_logw):
    ei = edge_index0.astype(jnp.int32).reshape(2, NW, STEPS, K)

    degp = _deg_kernel(ei)
    g1, dinv = _tc_pre(x0, conv1_w, degp.reshape(NC, NPAD, 1))
    acc1p = _segsum_kernel(g1, ei)
    w2p = jnp.pad(conv2_w, ((0, 0), (0, F - C)))
    p = _tc_mid(acc1p, g1, dinv, conv1_b.reshape(1, H), w2p)
    acc2p = _segsum_kernel(p, ei)
    cht = ch_logw.reshape(CH, N).T
    out = _tc_head(acc2p, p, dinv, conv2_b.reshape(1, C), cht, child1_w,
                   child1_b.reshape(1, C), child2_w, child2_b.reshape(1, C),
                   mlp_w, mlp_b.reshape(1, C), bn_gamma.reshape(1, C),
                   bn_beta.reshape(1, C))
    return out


# speedup vs baseline: 1.0647x; 1.0647x over previous
import functools

import jax
import jax.numpy as jnp
import numpy as np
from jax import lax
from jax.experimental import pallas as pl
from jax.experimental.pallas import tpu as pltpu
from jax.experimental.pallas import tpu_sc as plsc

N, E, F_IN, H, C, CH = 10000, 320000, 128, 16, 10, 3
NC, NS = 2, 16
NW = NC * NS
NPAD = 10240
ROWS_PER_SUB = NPAD // NS
F = 16
EPW = E // NW
K = 125
STEPS = EPW // K
NBUF = 8
STD = float(np.sqrt(2.0 / (N + 1)))

_mesh = plsc.VectorSubcoreMesh(core_axis_name="c", subcore_axis_name="s")


@functools.partial(
    pl.kernel,
    out_type=jax.ShapeDtypeStruct((NC, NPAD), jnp.float32),
    mesh=_mesh,
    scratch_types=[
        pltpu.VMEM((STEPS, K), jnp.int32),
        pltpu.VMEM((128,), jnp.float32),
        pltpu.VMEM((ROWS_PER_SUB,), jnp.float32),
        pltpu.VMEM_SHARED((NPAD,), jnp.float32),
        pltpu.SemaphoreType.DMA,
        [pltpu.SemaphoreType.DMA for _ in range(NBUF)],
    ],
    compiler_params=pltpu.CompilerParams(use_tc_tiling_on_sc=False),
)
def _deg_kernel(ei_hbm, out_hbm, didx, ones_v, zv, acc_s, sem, dsems):
    c = lax.axis_index("c")
    s = lax.axis_index("s")
    wid = s * NC + c

    ld = pltpu.async_copy(ei_hbm.at[1, wid], didx, sem)

    def fill_ones(i, carry):
        ones_v[pl.ds(i * 16, 16)] = jnp.ones((16,), jnp.float32)
        return carry

    lax.fori_loop(0, 128 // 16, fill_ones, 0)

    def fill_zero(i, carry):
        zv[pl.ds(i * 16, 16)] = jnp.zeros((16,), jnp.float32)
        return carry

    lax.fori_loop(0, ROWS_PER_SUB // 16, fill_zero, 0)
    pltpu.sync_copy(zv, acc_s.at[pl.ds(s * ROWS_PER_SUB, ROWS_PER_SUB)])
    ld.wait()
    plsc.subcore_barrier()

    def step(j, carry):
        sds = []
        for b in range(NBUF):
            sds.append(pltpu.async_copy(
                ones_v.at[pl.ds(0, K)], acc_s.at[didx.at[j * NBUF + b]],
                dsems[b], add=True))
        for b in range(NBUF):
            sds[b].wait()
        return carry

    lax.fori_loop(0, STEPS // NBUF, step, 0)
    plsc.subcore_barrier()
    pltpu.sync_copy(acc_s.at[pl.ds(s * ROWS_PER_SUB, ROWS_PER_SUB)],
                    out_hbm.at[c, pl.ds(s * ROWS_PER_SUB, ROWS_PER_SUB)])


@functools.partial(
    pl.kernel,
    out_type=jax.ShapeDtypeStruct((NC, NPAD, F), jnp.float32),
    mesh=_mesh,
    scratch_types=[
        pltpu.VMEM((2, STEPS, K), jnp.int32),
        [pltpu.VMEM((K, F), jnp.float32) for _ in range(NBUF)],
        pltpu.VMEM((ROWS_PER_SUB, F), jnp.float32),
        pltpu.VMEM_SHARED((NPAD, F), jnp.float32),
        pltpu.VMEM_SHARED((NPAD, F), jnp.float32),
        [pltpu.SemaphoreType.DMA for _ in range(NBUF)],
        [pltpu.SemaphoreType.DMA for _ in range(NBUF)],
        pltpu.SemaphoreType.DMA,
        pltpu.SemaphoreType.DMA,
    ],
    compiler_params=pltpu.CompilerParams(use_tc_tiling_on_sc=False),
)
def _segsum_kernel(g_hbm, ei_hbm, out_hbm,
                   eidx, rows, zv, acc_s, g_s, sems, ssems, isem, gsem):
    c = lax.axis_index("c")
    s = lax.axis_index("s")
    wid = s * NC + c

    ls = pltpu.async_copy(ei_hbm.at[0, wid], eidx.at[0], isem)
    row0 = s * ROWS_PER_SUB
    lg = pltpu.async_copy(g_hbm.at[pl.ds(row0, ROWS_PER_SUB)],
                          g_s.at[pl.ds(row0, ROWS_PER_SUB)], gsem)

    def fill_zero(i, carry):
        zv[i, :] = jnp.zeros((16,), jnp.float32)
        return carry

    lax.fori_loop(0, ROWS_PER_SUB, fill_zero, 0)
    pltpu.sync_copy(zv, acc_s.at[pl.ds(s * ROWS_PER_SUB, ROWS_PER_SUB)])
    ls.wait()
    pltpu.sync_copy(ei_hbm.at[1, wid], eidx.at[1])
    lg.wait()
    plsc.subcore_barrier()

    def batch(j, carry):
        gds, sds = [], []
        for b in range(NBUF):
            gds.append(pltpu.async_copy(
                g_s.at[eidx.at[0, j * NBUF + b]], rows[b], sems[b]))
        for b in range(NBUF):
            gds[b].wait()
            sds.append(pltpu.async_copy(
                rows[b], acc_s.at[eidx.at[1, j * NBUF + b]], ssems[b],
                add=True))
        for b in range(NBUF):
            sds[b].wait()
        return carry

    lax.fori_loop(0, STEPS // NBUF, batch, 0)
    plsc.subcore_barrier()
    pltpu.sync_copy(acc_s.at[pl.ds(s * ROWS_PER_SUB, ROWS_PER_SUB)],
                    out_hbm.at[c, pl.ds(s * ROWS_PER_SUB, ROWS_PER_SUB)])


def _tc_pre_body(x_ref, w1_ref, degp_ref, g_ref, dinv_ref):
    deg = degp_ref[0, :N, :] + degp_ref[1, :N, :] + 1.0
    dinv = lax.rsqrt(deg)
    dinv_ref[...] = dinv
    g_ref[:N, :] = jnp.dot(x_ref[...], w1_ref[...],
                           preferred_element_type=jnp.float32) * dinv
    g_ref[N:, :] = jnp.zeros((NPAD - N, F), jnp.float32)


def _tc_mid_body(accp_ref, g_ref, dinv_ref, b1_ref, w2p_ref, p_ref):
    acc = accp_ref[0, :N, :] + accp_ref[1, :N, :]
    dinv = dinv_ref[...]
    h = jnp.maximum(dinv * (acc + g_ref[:N, :]) + b1_ref[...], 0.0)
    p_ref[:N, :] = jnp.dot(h, w2p_ref[...],
                           preferred_element_type=jnp.float32) * dinv
    p_ref[N:, :] = jnp.zeros((NPAD - N, F), jnp.float32)


def _tc_head_body(accp_ref, p_ref, dinv_ref, b2_ref, cht_ref, c1w_ref,
                  c1b_ref, c2w_ref, c2b_ref, mlpw_ref, mlpb_ref, gam_ref,
                  bet_ref, out_ref):
    acc = accp_ref[0, :N, :] + accp_ref[1, :N, :]
    dinv = dinv_ref[...]
    z0 = (dinv * (acc + p_ref[:N, :]))[:, :C] + b2_ref[...]
    m = jnp.max(z0, axis=1, keepdims=True)
    lse = m + jnp.log(jnp.sum(jnp.exp(z0 - m), axis=1, keepdims=True))
    h0 = z0 - lse
    w0 = jnp.exp(cht_ref[:, 0:1] * STD)
    w1 = jnp.exp(cht_ref[:, 1:2] * STD)
    w2 = jnp.exp(cht_ref[:, 2:3] * STD)
    o0 = w0 * h0
    o1 = w1 * (jnp.dot(o0, c1w_ref[...],
                       preferred_element_type=jnp.float32) + c1b_ref[...])
    o01 = jnp.concatenate([o0, o1], axis=1)
    o2 = w2 * (jnp.dot(o01, c2w_ref[...],
                       preferred_element_type=jnp.float32) + c2b_ref[...])
    zc = jnp.concatenate([o01, o2], axis=1)
    z = jnp.dot(zc, mlpw_ref[...],
                preferred_element_type=jnp.float32) + mlpb_ref[...]
    mu = jnp.mean(z, axis=0, keepdims=True)
    var = jnp.mean((z - mu) ** 2, axis=0, keepdims=True)
    zn = (z - mu) * lax.rsqrt(var + 1e-5) * gam_ref[...] + bet_ref[...]
    m2 = jnp.max(zn, axis=1, keepdims=True)
    lse2 = m2 + jnp.log(jnp.sum(jnp.exp(zn - m2), axis=1, keepdims=True))
    out_ref[...] = zn - lse2


_tc_pre = pl.pallas_call(
    _tc_pre_body,
    out_shape=[jax.ShapeDtypeStruct((NPAD, H), jnp.float32),
               jax.ShapeDtypeStruct((N, 1), jnp.float32)],
)

_tc_mid = pl.pallas_call(
    _tc_mid_body,
    out_shape=jax.ShapeDtypeStruct((NPAD, F), jnp.float32),
)

_tc_head = pl.pallas_call(
    _tc_head_body,
    out_shape=jax.ShapeDtypeStruct((N, C), jnp.float32),
)


def kernel(x0, x1, x2, edge_index0, edge_index1, edge_index2, conv1_w,
           conv1_b, conv2_w, conv2_b, child1_w, child1_b, child2_w,
           child2_b, mlp_w, mlp_b, bn_gamma, bn_beta, ch_logw):
    ei = edge_index0.astype(jnp.int32).reshape(2, NW, STEPS, K)

    degp = _deg_kernel(ei)
    g1, dinv = _tc_pre(x0, conv1_w, degp.reshape(NC, NPAD, 1))
    acc1p = _segsum_kernel(g1, ei)
    w2p = jnp.pad(conv2_w, ((0, 0), (0, F - C)))
    p = _tc_mid(acc1p, g1, dinv, conv1_b.reshape(1, H), w2p)
    acc2p = _segsum_kernel(p, ei)
    cht = ch_logw.reshape(CH, N).T
    out = _tc_head(acc2p, p, dinv, conv2_b.reshape(1, C), cht, child1_w,
                   child1_b.reshape(1, C), child2_w, child2_b.reshape(1, C),
                   mlp_w, mlp_b.reshape(1, C), bn_gamma.reshape(1, C),
                   bn_beta.reshape(1, C))
    return out


# speedup vs baseline: 1.0775x; 1.0120x over previous
import functools

import jax
import jax.numpy as jnp
import numpy as np
from jax import lax
from jax.experimental import pallas as pl
from jax.experimental.pallas import tpu as pltpu
from jax.experimental.pallas import tpu_sc as plsc

N, E, F_IN, H, C, CH = 10000, 320000, 128, 16, 10, 3
NC, NS = 2, 16
NW = NC * NS
NPAD = 10240
ROWS_PER_SUB = NPAD // NS
F = 16
EPW = E // NW
K = 128
STEPS = 80
EPWP = STEPS * K
NBUF = 8
STD = float(np.sqrt(2.0 / (N + 1)))

_mesh = plsc.VectorSubcoreMesh(core_axis_name="c", subcore_axis_name="s")


@functools.partial(
    pl.kernel,
    out_type=jax.ShapeDtypeStruct((NC, NPAD), jnp.float32),
    mesh=_mesh,
    scratch_types=[
        pltpu.VMEM((STEPS, K), jnp.int32),
        pltpu.VMEM((128,), jnp.float32),
        pltpu.VMEM((ROWS_PER_SUB,), jnp.float32),
        pltpu.VMEM_SHARED((NPAD,), jnp.float32),
        pltpu.SemaphoreType.DMA,
        [pltpu.SemaphoreType.DMA for _ in range(NBUF)],
    ],
    compiler_params=pltpu.CompilerParams(use_tc_tiling_on_sc=False),
)
def _deg_kernel(ei_hbm, out_hbm, didx, ones_v, zv, acc_s, sem, dsems):
    c = lax.axis_index("c")
    s = lax.axis_index("s")
    wid = s * NC + c

    ld = pltpu.async_copy(ei_hbm.at[1, wid], didx, sem)

    def fill_ones(i, carry):
        ones_v[pl.ds(i * 16, 16)] = jnp.ones((16,), jnp.float32)
        return carry

    lax.fori_loop(0, 128 // 16, fill_ones, 0)

    def fill_zero(i, carry):
        zv[pl.ds(i * 16, 16)] = jnp.zeros((16,), jnp.float32)
        return carry

    lax.fori_loop(0, ROWS_PER_SUB // 16, fill_zero, 0)
    pltpu.sync_copy(zv, acc_s.at[pl.ds(s * ROWS_PER_SUB, ROWS_PER_SUB)])
    ld.wait()
    plsc.subcore_barrier()

    def step(j, carry):
        sds = []
        for b in range(NBUF):
            sds.append(pltpu.async_copy(
                ones_v, acc_s.at[didx.at[j * NBUF + b]],
                dsems[b], add=True))
        for b in range(NBUF):
            sds[b].wait()
        return carry

    lax.fori_loop(0, STEPS // NBUF, step, 0)
    plsc.subcore_barrier()
    pltpu.sync_copy(acc_s.at[pl.ds(s * ROWS_PER_SUB, ROWS_PER_SUB)],
                    out_hbm.at[c, pl.ds(s * ROWS_PER_SUB, ROWS_PER_SUB)])


@functools.partial(
    pl.kernel,
    out_type=jax.ShapeDtypeStruct((NC, NPAD, F), jnp.float32),
    mesh=_mesh,
    scratch_types=[
        pltpu.VMEM((2, STEPS, K), jnp.int32),
        [pltpu.VMEM((K, F), jnp.float32) for _ in range(NBUF)],
        pltpu.VMEM((ROWS_PER_SUB, F), jnp.float32),
        pltpu.VMEM_SHARED((NPAD, F), jnp.float32),
        pltpu.VMEM_SHARED((NPAD, F), jnp.float32),
        [pltpu.SemaphoreType.DMA for _ in range(NBUF)],
        [pltpu.SemaphoreType.DMA for _ in range(NBUF)],
        pltpu.SemaphoreType.DMA,
        pltpu.SemaphoreType.DMA,
    ],
    compiler_params=pltpu.CompilerParams(use_tc_tiling_on_sc=False),
)
def _segsum_kernel(g_hbm, ei_hbm, out_hbm,
                   eidx, rows, zv, acc_s, g_s, sems, ssems, isem, gsem):
    c = lax.axis_index("c")
    s = lax.axis_index("s")
    wid = s * NC + c

    ls = pltpu.async_copy(ei_hbm.at[0, wid], eidx.at[0], isem)
    row0 = s * ROWS_PER_SUB
    lg = pltpu.async_copy(g_hbm.at[pl.ds(row0, ROWS_PER_SUB)],
                          g_s.at[pl.ds(row0, ROWS_PER_SUB)], gsem)

    def fill_zero(i, carry):
        zv[i, :] = jnp.zeros((16,), jnp.float32)
        return carry

    lax.fori_loop(0, ROWS_PER_SUB, fill_zero, 0)
    pltpu.sync_copy(zv, acc_s.at[pl.ds(s * ROWS_PER_SUB, ROWS_PER_SUB)])
    ls.wait()
    pltpu.sync_copy(ei_hbm.at[1, wid], eidx.at[1])
    lg.wait()
    plsc.subcore_barrier()

    def batch(j, carry):
        gds, sds = [], []
        for b in range(NBUF):
            gds.append(pltpu.async_copy(
                g_s.at[eidx.at[0, j * NBUF + b]], rows[b], sems[b]))
        for b in range(NBUF):
            gds[b].wait()
            sds.append(pltpu.async_copy(
                rows[b], acc_s.at[eidx.at[1, j * NBUF + b]], ssems[b],
                add=True))
        for b in range(NBUF):
            sds[b].wait()
        return carry

    lax.fori_loop(0, STEPS // NBUF, batch, 0)
    plsc.subcore_barrier()
    pltpu.sync_copy(acc_s.at[pl.ds(s * ROWS_PER_SUB, ROWS_PER_SUB)],
                    out_hbm.at[c, pl.ds(s * ROWS_PER_SUB, ROWS_PER_SUB)])


def _tc_pre_body(x_ref, w1_ref, degp_ref, g_ref, dinv_ref):
    deg = degp_ref[0, :N, :] + degp_ref[1, :N, :] + 1.0
    dinv = lax.rsqrt(deg)
    dinv_ref[...] = dinv
    g_ref[:N, :] = jnp.dot(x_ref[...], w1_ref[...],
                           preferred_element_type=jnp.float32) * dinv
    g_ref[N:, :] = jnp.zeros((NPAD - N, F), jnp.float32)


def _tc_mid_body(accp_ref, g_ref, dinv_ref, b1_ref, w2p_ref, p_ref):
    acc = accp_ref[0, :N, :] + accp_ref[1, :N, :]
    dinv = dinv_ref[...]
    h = jnp.maximum(dinv * (acc + g_ref[:N, :]) + b1_ref[...], 0.0)
    p_ref[:N, :] = jnp.dot(h, w2p_ref[...],
                           preferred_element_type=jnp.float32) * dinv
    p_ref[N:, :] = jnp.zeros((NPAD - N, F), jnp.float32)


def _tc_head_body(accp_ref, p_ref, dinv_ref, b2_ref, cht_ref, c1w_ref,
                  c1b_ref, c2w_ref, c2b_ref, mlpw_ref, mlpb_ref, gam_ref,
                  bet_ref, out_ref):
    acc = accp_ref[0, :N, :] + accp_ref[1, :N, :]
    dinv = dinv_ref[...]
    z0 = (dinv * (acc + p_ref[:N, :]))[:, :C] + b2_ref[...]
    m = jnp.max(z0, axis=1, keepdims=True)
    lse = m + jnp.log(jnp.sum(jnp.exp(z0 - m), axis=1, keepdims=True))
    h0 = z0 - lse
    w0 = jnp.exp(cht_ref[:, 0:1] * STD)
    w1 = jnp.exp(cht_ref[:, 1:2] * STD)
    w2 = jnp.exp(cht_ref[:, 2:3] * STD)
    o0 = w0 * h0
    o1 = w1 * (jnp.dot(o0, c1w_ref[...],
                       preferred_element_type=jnp.float32) + c1b_ref[...])
    o01 = jnp.concatenate([o0, o1], axis=1)
    o2 = w2 * (jnp.dot(o01, c2w_ref[...],
                       preferred_element_type=jnp.float32) + c2b_ref[...])
    zc = jnp.concatenate([o01, o2], axis=1)
    z = jnp.dot(zc, mlpw_ref[...],
                preferred_element_type=jnp.float32) + mlpb_ref[...]
    mu = jnp.mean(z, axis=0, keepdims=True)
    var = jnp.mean((z - mu) ** 2, axis=0, keepdims=True)
    zn = (z - mu) * lax.rsqrt(var + 1e-5) * gam_ref[...] + bet_ref[...]
    m2 = jnp.max(zn, axis=1, keepdims=True)
    lse2 = m2 + jnp.log(jnp.sum(jnp.exp(zn - m2), axis=1, keepdims=True))
    out_ref[...] = zn - lse2


_tc_pre = pl.pallas_call(
    _tc_pre_body,
    out_shape=[jax.ShapeDtypeStruct((NPAD, H), jnp.float32),
               jax.ShapeDtypeStruct((N, 1), jnp.float32)],
)

_tc_mid = pl.pallas_call(
    _tc_mid_body,
    out_shape=jax.ShapeDtypeStruct((NPAD, F), jnp.float32),
)

_tc_head = pl.pallas_call(
    _tc_head_body,
    out_shape=jax.ShapeDtypeStruct((N, C), jnp.float32),
)


def kernel(x0, x1, x2, edge_index0, edge_index1, edge_index2, conv1_w,
           conv1_b, conv2_w, conv2_b, child1_w, child1_b, child2_w,
           child2_b, mlp_w, mlp_b, bn_gamma, bn_beta, ch_logw):
    ei = edge_index0.astype(jnp.int32).reshape(2, NW, EPW)
    padv = N + jnp.arange(EPWP - EPW, dtype=jnp.int32) % (NPAD - N)
    padv = jnp.broadcast_to(padv, (2, NW, EPWP - EPW))
    ei = jnp.concatenate([ei, padv], axis=2).reshape(2, NW, STEPS, K)

    degp = _deg_kernel(ei)
    g1, dinv = _tc_pre(x0, conv1_w, degp.reshape(NC, NPAD, 1))
    acc1p = _segsum_kernel(g1, ei)
    w2p = jnp.pad(conv2_w, ((0, 0), (0, F - C)))
    p = _tc_mid(acc1p, g1, dinv, conv1_b.reshape(1, H), w2p)
    acc2p = _segsum_kernel(p, ei)
    cht = ch_logw.reshape(CH, N).T
    out = _tc_head(acc2p, p, dinv, conv2_b.reshape(1, C), cht, child1_w,
                   child1_b.reshape(1, C), child2_w, child2_b.reshape(1, C),
                   mlp_w, mlp_b.reshape(1, C), bn_gamma.reshape(1, C),
                   bn_beta.reshape(1, C))
    return out
